# baseline (speedup 1.0000x reference)
import functools

import jax
import jax.numpy as jnp
from jax import lax
from jax.experimental import pallas as pl
from jax.experimental.pallas import tpu as pltpu
from jax.experimental.pallas import tpu_sc as plsc

_VOCAB_TILE = 2048


@functools.cache
def _sc_gather(num_rows: int, feat: int):
    info = plsc.get_sparse_core_info()
    nc, ns = info.num_cores, info.num_subcores
    nw = nc * ns
    assert num_rows % (8 * nw) == 0 and feat % info.num_lanes == 0
    rows_per_w = num_rows // nw
    mesh = plsc.VectorSubcoreMesh(core_axis_name="c", subcore_axis_name="s")

    @functools.partial(
        pl.kernel,
        mesh=mesh,
        out_type=jax.ShapeDtypeStruct((num_rows, feat), jnp.float32),
        scratch_types=[
            pltpu.VMEM((rows_per_w,), jnp.int32),
            pltpu.VMEM((rows_per_w, feat), jnp.float32),
            pltpu.SemaphoreType.DMA,
        ],
    )
    def gather(table_hbm, idx_hbm, out_hbm, idx_v, rows_v, sem):
        wid = lax.axis_index("s") * nc + lax.axis_index("c")
        base = wid * rows_per_w
        pltpu.sync_copy(idx_hbm.at[pl.ds(base, rows_per_w)], idx_v)
        pltpu.async_copy(table_hbm.at[idx_v], rows_v, sem).wait()
        pltpu.sync_copy(rows_v, out_hbm.at[pl.ds(base, rows_per_w)])

    return gather


def _mlp_body(emb_ref, w1_ref, b1_ref, w2_ref, b2_ref, w3_ref, b3_ref,
              out_ref, h2_ref):
    @pl.when(pl.program_id(0) == 0)
    def _():
        h1 = jnp.maximum(
            jnp.dot(emb_ref[...], w1_ref[...],
                    preferred_element_type=jnp.float32) + b1_ref[...], 0.0)
        h2_ref[...] = jnp.maximum(
            jnp.dot(h1, w2_ref[...],
                    preferred_element_type=jnp.float32) + b2_ref[...], 0.0)

    out_ref[...] = jnp.dot(h2_ref[...], w3_ref[...],
                           preferred_element_type=jnp.float32) + b3_ref[...]


def kernel(x, table, W1, b1, W2, b2, W3, b3):
    b, k = x.shape
    v, h = table.shape
    p = W3.shape[1]
    d_in = k * h

    idx = x.reshape(-1).astype(jnp.int32)
    rows = _sc_gather(b * k, h)(table, idx)
    emb = rows.reshape(b, d_in)

    pt = _VOCAB_TILE
    grid = (pl.cdiv(p, pt),)
    out = pl.pallas_call(
        _mlp_body,
        grid=grid,
        in_specs=[
            pl.BlockSpec((b, d_in), lambda i: (0, 0)),
            pl.BlockSpec((d_in, h), lambda i: (0, 0)),
            pl.BlockSpec((1, h), lambda i: (0, 0)),
            pl.BlockSpec((h, h), lambda i: (0, 0)),
            pl.BlockSpec((1, h), lambda i: (0, 0)),
            pl.BlockSpec((h, pt), lambda i: (0, i)),
            pl.BlockSpec((1, pt), lambda i: (0, i)),
        ],
        out_specs=pl.BlockSpec((b, pt), lambda i: (0, i)),
        out_shape=jax.ShapeDtypeStruct((b, p), jnp.float32),
        scratch_shapes=[pltpu.VMEM((b, h), jnp.float32)],
    )(emb, W1, b1.reshape(1, h), W2, b2.reshape(1, h), W3, b3.reshape(1, p))
    return out


# speedup vs baseline: 3.3428x; 3.3428x over previous
import functools

import jax
import jax.numpy as jnp
from jax import lax
from jax.experimental import pallas as pl
from jax.experimental.pallas import tpu as pltpu
from jax.experimental.pallas import tpu_sc as plsc

_VOCAB_TILE = 2000


@functools.cache
def _sc_gather(num_rows: int, feat: int):
    info = plsc.get_sparse_core_info()
    nc, ns = info.num_cores, info.num_subcores
    nw = nc * ns
    assert num_rows % (8 * nw) == 0 and feat % info.num_lanes == 0
    rows_per_w = num_rows // nw
    mesh = plsc.VectorSubcoreMesh(core_axis_name="c", subcore_axis_name="s")

    @functools.partial(
        pl.kernel,
        mesh=mesh,
        out_type=jax.ShapeDtypeStruct((num_rows, feat), jnp.float32),
        scratch_types=[
            pltpu.VMEM((rows_per_w,), jnp.int32),
            pltpu.VMEM((rows_per_w, feat), jnp.float32),
            pltpu.SemaphoreType.DMA,
        ],
    )
    def gather(table_hbm, idx_hbm, out_hbm, idx_v, rows_v, sem):
        wid = lax.axis_index("s") * nc + lax.axis_index("c")
        base = wid * rows_per_w
        pltpu.sync_copy(idx_hbm.at[pl.ds(base, rows_per_w)], idx_v)
        pltpu.async_copy(table_hbm.at[idx_v], rows_v, sem).wait()
        pltpu.sync_copy(rows_v, out_hbm.at[pl.ds(base, rows_per_w)])

    return gather


def _mlp_body(emb_ref, w1_ref, b1_ref, w2_ref, b2_ref, w3t_ref, b3_ref,
              out_ref, h2t_ref):
    @pl.when(pl.program_id(0) == 0)
    def _():
        h1 = jnp.maximum(
            jnp.dot(emb_ref[...], w1_ref[...],
                    preferred_element_type=jnp.float32) + b1_ref[...], 0.0)
        h2 = jnp.maximum(
            jnp.dot(h1, w2_ref[...],
                    preferred_element_type=jnp.float32) + b2_ref[...], 0.0)
        h2t_ref[...] = h2.T

    out_ref[...] = (
        jnp.dot(w3t_ref[...], h2t_ref[...],
                preferred_element_type=jnp.float32)
        + b3_ref[0].T)


def kernel(x, table, W1, b1, W2, b2, W3, b3):
    b, k = x.shape
    v, h = table.shape
    p = W3.shape[1]
    d_in = k * h

    idx = x.reshape(-1).astype(jnp.int32)
    rows = _sc_gather(b * k, h)(table, idx)
    emb = rows.reshape(b, d_in)

    pt = _VOCAB_TILE
    grid = (pl.cdiv(p, pt),)
    out_t = pl.pallas_call(
        _mlp_body,
        grid=grid,
        in_specs=[
            pl.BlockSpec((b, d_in), lambda i: (0, 0)),
            pl.BlockSpec((d_in, h), lambda i: (0, 0)),
            pl.BlockSpec((1, h), lambda i: (0, 0)),
            pl.BlockSpec((h, h), lambda i: (0, 0)),
            pl.BlockSpec((1, h), lambda i: (0, 0)),
            pl.BlockSpec((pt, h), lambda i: (i, 0)),
            pl.BlockSpec((1, 1, pt), lambda i: (i, 0, 0)),
        ],
        out_specs=pl.BlockSpec((pt, b), lambda i: (i, 0)),
        out_shape=jax.ShapeDtypeStruct((p, b), jnp.float32),
        scratch_shapes=[pltpu.VMEM((h, b), jnp.float32)],
    )(emb, W1, b1.reshape(1, h), W2, b2.reshape(1, h), W3.T,
      b3.reshape(p // pt, 1, pt))
    return out_t.T


# speedup vs baseline: 3.4097x; 1.0200x over previous
import functools

import jax
import jax.numpy as jnp
from jax import lax
from jax.experimental import pallas as pl
from jax.experimental.pallas import tpu as pltpu
from jax.experimental.pallas import tpu_sc as plsc

_VOCAB_TILE = 2000


@functools.cache
def _sc_gather(num_rows: int, feat: int):
    info = plsc.get_sparse_core_info()
    nc, ns = info.num_cores, info.num_subcores
    nw = nc * ns
    assert num_rows % (8 * nw) == 0 and feat % info.num_lanes == 0
    rows_per_w = num_rows // nw
    mesh = plsc.VectorSubcoreMesh(core_axis_name="c", subcore_axis_name="s")

    @functools.partial(
        pl.kernel,
        mesh=mesh,
        out_type=jax.ShapeDtypeStruct((num_rows, feat), jnp.float32),
        scratch_types=[
            pltpu.VMEM((rows_per_w,), jnp.int32),
            pltpu.VMEM((rows_per_w, feat), jnp.float32),
            pltpu.SemaphoreType.DMA,
        ],
    )
    def gather(table_hbm, idx_hbm, out_hbm, idx_v, rows_v, sem):
        wid = lax.axis_index("s") * nc + lax.axis_index("c")
        base = wid * rows_per_w
        pltpu.sync_copy(idx_hbm.at[pl.ds(base, rows_per_w)], idx_v)
        pltpu.async_copy(table_hbm.at[idx_v], rows_v, sem).wait()
        pltpu.sync_copy(rows_v, out_hbm.at[pl.ds(base, rows_per_w)])

    return gather


def _mlp_body(rows_ref, w1_ref, b1_ref, w2_ref, b2_ref, w3t_ref, b3_ref,
              out_ref, h2t_ref):
    @pl.when(pl.program_id(0) == 0)
    def _():
        n = rows_ref.shape[0] // 2
        h = rows_ref.shape[1]
        h1 = jnp.maximum(
            jnp.dot(rows_ref[:n], w1_ref[:h],
                    preferred_element_type=jnp.float32)
            + jnp.dot(rows_ref[n:], w1_ref[h:],
                      preferred_element_type=jnp.float32)
            + b1_ref[...], 0.0)
        h2 = jnp.maximum(
            jnp.dot(h1, w2_ref[...],
                    preferred_element_type=jnp.float32) + b2_ref[...], 0.0)
        h2t_ref[...] = h2.T

    out_ref[...] = (
        jnp.dot(w3t_ref[...], h2t_ref[...],
                preferred_element_type=jnp.float32)
        + b3_ref[0].T)


def kernel(x, table, W1, b1, W2, b2, W3, b3):
    b, k = x.shape
    v, h = table.shape
    p = W3.shape[1]
    d_in = k * h

    idx = x.T.reshape(-1).astype(jnp.int32)
    rows = _sc_gather(b * k, h)(table, idx)

    pt = _VOCAB_TILE
    grid = (pl.cdiv(p, pt),)
    out_t = pl.pallas_call(
        _mlp_body,
        grid=grid,
        in_specs=[
            pl.BlockSpec((b * k, h), lambda i: (0, 0)),
            pl.BlockSpec((d_in, h), lambda i: (0, 0)),
            pl.BlockSpec((1, h), lambda i: (0, 0)),
            pl.BlockSpec((h, h), lambda i: (0, 0)),
            pl.BlockSpec((1, h), lambda i: (0, 0)),
            pl.BlockSpec((pt, h), lambda i: (i, 0)),
            pl.BlockSpec((1, 1, pt), lambda i: (i, 0, 0)),
        ],
        out_specs=pl.BlockSpec((pt, b), lambda i: (i, 0)),
        out_shape=jax.ShapeDtypeStruct((p, b), jnp.float32),
        scratch_shapes=[pltpu.VMEM((h, b), jnp.float32)],
    )(rows, W1, b1.reshape(1, h), W2, b2.reshape(1, h), W3.T,
      b3.reshape(p // pt, 1, pt))
    return out_t.T


# speedup vs baseline: 3.4521x; 1.0125x over previous
import functools

import jax
import jax.numpy as jnp
from jax import lax
from jax.experimental import pallas as pl
from jax.experimental.pallas import tpu as pltpu
from jax.experimental.pallas import tpu_sc as plsc

_VOCAB_TILE = 4000


@functools.cache
def _sc_gather(num_rows: int, feat: int):
    info = plsc.get_sparse_core_info()
    nc, ns = info.num_cores, info.num_subcores
    nw = nc * ns
    assert num_rows % (8 * nw) == 0 and feat % info.num_lanes == 0
    rows_per_w = num_rows // nw
    mesh = plsc.VectorSubcoreMesh(core_axis_name="c", subcore_axis_name="s")

    @functools.partial(
        pl.kernel,
        mesh=mesh,
        out_type=jax.ShapeDtypeStruct((num_rows, feat), jnp.float32),
        scratch_types=[
            pltpu.VMEM((rows_per_w,), jnp.int32),
            pltpu.VMEM((rows_per_w, feat), jnp.float32),
            pltpu.SemaphoreType.DMA,
        ],
    )
    def gather(table_hbm, idx_hbm, out_hbm, idx_v, rows_v, sem):
        wid = lax.axis_index("s") * nc + lax.axis_index("c")
        base = wid * rows_per_w
        pltpu.sync_copy(idx_hbm.at[pl.ds(base, rows_per_w)], idx_v)
        pltpu.async_copy(table_hbm.at[idx_v], rows_v, sem).wait()
        pltpu.sync_copy(rows_v, out_hbm.at[pl.ds(base, rows_per_w)])

    return gather


def _mlp_body(rows_ref, w1_ref, b1_ref, w2_ref, b2_ref, w3t_ref, b3_ref,
              out_ref, h2t_ref):
    @pl.when(pl.program_id(0) == 0)
    def _():
        n = rows_ref.shape[0] // 2
        h = rows_ref.shape[1]
        h1 = jnp.maximum(
            jnp.dot(rows_ref[:n], w1_ref[:h],
                    preferred_element_type=jnp.float32)
            + jnp.dot(rows_ref[n:], w1_ref[h:],
                      preferred_element_type=jnp.float32)
            + b1_ref[...], 0.0)
        h2 = jnp.maximum(
            jnp.dot(h1, w2_ref[...],
                    preferred_element_type=jnp.float32) + b2_ref[...], 0.0)
        h2t_ref[...] = h2.T

    out_ref[...] = (
        jnp.dot(w3t_ref[...], h2t_ref[...],
                preferred_element_type=jnp.float32)
        + b3_ref[0].T)


def kernel(x, table, W1, b1, W2, b2, W3, b3):
    b, k = x.shape
    v, h = table.shape
    p = W3.shape[1]
    d_in = k * h

    idx = x.T.reshape(-1).astype(jnp.int32)
    rows = _sc_gather(b * k, h)(table, idx)

    pt = _VOCAB_TILE
    grid = (pl.cdiv(p, pt),)
    out_t = pl.pallas_call(
        _mlp_body,
        grid=grid,
        in_specs=[
            pl.BlockSpec((b * k, h), lambda i: (0, 0)),
            pl.BlockSpec((d_in, h), lambda i: (0, 0)),
            pl.BlockSpec((1, h), lambda i: (0, 0)),
            pl.BlockSpec((h, h), lambda i: (0, 0)),
            pl.BlockSpec((1, h), lambda i: (0, 0)),
            pl.BlockSpec((pt, h), lambda i: (i, 0)),
            pl.BlockSpec((1, 1, pt), lambda i: (i, 0, 0)),
        ],
        out_specs=pl.BlockSpec((pt, b), lambda i: (i, 0)),
        out_shape=jax.ShapeDtypeStruct((p, b), jnp.float32),
        scratch_shapes=[pltpu.VMEM((h, b), jnp.float32)],
    )(rows, W1, b1.reshape(1, h), W2, b2.reshape(1, h), W3.T,
      b3.reshape(p // pt, 1, pt))
    return out_t.T


# speedup vs baseline: 3.4555x; 1.0010x over previous
import functools

import jax
import jax.numpy as jnp
from jax import lax
from jax.experimental import pallas as pl
from jax.experimental.pallas import tpu as pltpu
from jax.experimental.pallas import tpu_sc as plsc

_VOCAB_TILE = 5000


@functools.cache
def _sc_gather(num_rows: int, feat: int):
    info = plsc.get_sparse_core_info()
    nc, ns = info.num_cores, info.num_subcores
    nw = nc * ns
    assert num_rows % (8 * nw) == 0 and feat % info.num_lanes == 0
    rows_per_w = num_rows // nw
    mesh = plsc.VectorSubcoreMesh(core_axis_name="c", subcore_axis_name="s")

    @functools.partial(
        pl.kernel,
        mesh=mesh,
        out_type=jax.ShapeDtypeStruct((num_rows, feat), jnp.float32),
        scratch_types=[
            pltpu.VMEM((rows_per_w,), jnp.int32),
            pltpu.VMEM((rows_per_w, feat), jnp.float32),
            pltpu.SemaphoreType.DMA,
        ],
    )
    def gather(table_hbm, idx_hbm, out_hbm, idx_v, rows_v, sem):
        wid = lax.axis_index("s") * nc + lax.axis_index("c")
        base = wid * rows_per_w
        pltpu.sync_copy(idx_hbm.at[pl.ds(base, rows_per_w)], idx_v)
        pltpu.async_copy(table_hbm.at[idx_v], rows_v, sem).wait()
        pltpu.sync_copy(rows_v, out_hbm.at[pl.ds(base, rows_per_w)])

    return gather


def _mlp_body(rows_ref, w1_ref, b1_ref, w2_ref, b2_ref, w3t_ref, b3_ref,
              out_ref, h2t_ref):
    @pl.when(pl.program_id(0) == 0)
    def _():
        n = rows_ref.shape[0] // 2
        h = rows_ref.shape[1]
        h1 = jnp.maximum(
            jnp.dot(rows_ref[:n], w1_ref[:h],
                    preferred_element_type=jnp.float32)
            + jnp.dot(rows_ref[n:], w1_ref[h:],
                      preferred_element_type=jnp.float32)
            + b1_ref[...], 0.0)
        h2 = jnp.maximum(
            jnp.dot(h1, w2_ref[...],
                    preferred_element_type=jnp.float32) + b2_ref[...], 0.0)
        h2t_ref[...] = h2.T

    out_ref[...] = (
        jnp.dot(w3t_ref[...], h2t_ref[...],
                preferred_element_type=jnp.float32)
        + b3_ref[0].T)


def kernel(x, table, W1, b1, W2, b2, W3, b3):
    b, k = x.shape
    v, h = table.shape
    p = W3.shape[1]
    d_in = k * h

    idx = x.T.reshape(-1).astype(jnp.int32)
    rows = _sc_gather(b * k, h)(table, idx)

    pt = _VOCAB_TILE
    grid = (pl.cdiv(p, pt),)
    out_t = pl.pallas_call(
        _mlp_body,
        grid=grid,
        in_specs=[
            pl.BlockSpec((b * k, h), lambda i: (0, 0)),
            pl.BlockSpec((d_in, h), lambda i: (0, 0)),
            pl.BlockSpec((1, h), lambda i: (0, 0)),
            pl.BlockSpec((h, h), lambda i: (0, 0)),
            pl.BlockSpec((1, h), lambda i: (0, 0)),
            pl.BlockSpec((pt, h), lambda i: (i, 0)),
            pl.BlockSpec((1, 1, pt), lambda i: (i, 0, 0)),
        ],
        out_specs=pl.BlockSpec((pt, b), lambda i: (i, 0)),
        out_shape=jax.ShapeDtypeStruct((p, b), jnp.float32),
        scratch_shapes=[pltpu.VMEM((h, b), jnp.float32)],
    )(rows, W1, b1.reshape(1, h), W2, b2.reshape(1, h), W3.T,
      b3.reshape(p // pt, 1, pt))
    return out_t.T


# speedup vs baseline: 3.5038x; 1.0140x over previous
import functools

import jax
import jax.numpy as jnp
from jax import lax
from jax.experimental import pallas as pl
from jax.experimental.pallas import tpu as pltpu
from jax.experimental.pallas import tpu_sc as plsc

_VOCAB_TILE = 4096


@functools.cache
def _sc_gather(num_rows: int, feat: int):
    info = plsc.get_sparse_core_info()
    nc, ns = info.num_cores, info.num_subcores
    nw = nc * ns
    assert num_rows % (8 * nw) == 0 and feat % info.num_lanes == 0
    rows_per_w = num_rows // nw
    mesh = plsc.VectorSubcoreMesh(core_axis_name="c", subcore_axis_name="s")

    @functools.partial(
        pl.kernel,
        mesh=mesh,
        out_type=jax.ShapeDtypeStruct((num_rows, feat), jnp.float32),
        scratch_types=[
            pltpu.VMEM((rows_per_w,), jnp.int32),
            pltpu.VMEM((rows_per_w, feat), jnp.float32),
            pltpu.SemaphoreType.DMA,
        ],
    )
    def gather(table_hbm, idx_hbm, out_hbm, idx_v, rows_v, sem):
        wid = lax.axis_index("s") * nc + lax.axis_index("c")
        base = wid * rows_per_w
        pltpu.sync_copy(idx_hbm.at[pl.ds(base, rows_per_w)], idx_v)
        pltpu.async_copy(table_hbm.at[idx_v], rows_v, sem).wait()
        pltpu.sync_copy(rows_v, out_hbm.at[pl.ds(base, rows_per_w)])

    return gather


def _mlp_body(rows_ref, w1_ref, b1_ref, w2_ref, b2_ref, w3t_ref, b3_ref,
              out_ref, h2t_ref):
    @pl.when(pl.program_id(0) == 0)
    def _():
        n = rows_ref.shape[0] // 2
        h = rows_ref.shape[1]
        h1 = jnp.maximum(
            jnp.dot(rows_ref[:n], w1_ref[:h],
                    preferred_element_type=jnp.float32)
            + jnp.dot(rows_ref[n:], w1_ref[h:],
                      preferred_element_type=jnp.float32)
            + b1_ref[...], 0.0)
        h2 = jnp.maximum(
            jnp.dot(h1, w2_ref[...],
                    preferred_element_type=jnp.float32) + b2_ref[...], 0.0)
        h2t_ref[...] = h2.T

    out_ref[...] = (
        jnp.dot(w3t_ref[...], h2t_ref[...],
                preferred_element_type=jnp.float32)
        + b3_ref[...].T)


def kernel(x, table, W1, b1, W2, b2, W3, b3):
    b, k = x.shape
    v, h = table.shape
    p = W3.shape[1]
    d_in = k * h

    idx = x.T.reshape(-1).astype(jnp.int32)
    rows = _sc_gather(b * k, h)(table, idx)

    pt = _VOCAB_TILE
    grid = (pl.cdiv(p, pt),)
    out_t = pl.pallas_call(
        _mlp_body,
        grid=grid,
        in_specs=[
            pl.BlockSpec((b * k, h), lambda i: (0, 0)),
            pl.BlockSpec((d_in, h), lambda i: (0, 0)),
            pl.BlockSpec((1, h), lambda i: (0, 0)),
            pl.BlockSpec((h, h), lambda i: (0, 0)),
            pl.BlockSpec((1, h), lambda i: (0, 0)),
            pl.BlockSpec((pt, h), lambda i: (i, 0)),
            pl.BlockSpec((1, pt), lambda i: (0, i)),
        ],
        out_specs=pl.BlockSpec((pt, b), lambda i: (i, 0)),
        out_shape=jax.ShapeDtypeStruct((p, b), jnp.float32),
        scratch_shapes=[pltpu.VMEM((h, b), jnp.float32)],
    )(rows, W1, b1.reshape(1, h), W2, b2.reshape(1, h), W3.T,
      b3.reshape(1, p))
    return out_t.T
